# baseline (speedup 1.0000x reference)
import numpy as np

import jax
import jax.numpy as jnp
from jax.experimental import pallas as pl
from jax.experimental.pallas import tpu as pltpu


_DT = jnp.bfloat16


def _round_up(v, m):
    return (v + m - 1) // m * m


def _lenet_body(x_ref, wg_ref, b1_ref, w3_ref, b3_ref, w5_ref, bo_ref,
                out_ref, a1_s, a3_s):
    f32 = jnp.float32
    parts = [
        jnp.dot(x_ref[:, 128 * g:128 * g + 256], wg_ref[...],
                preferred_element_type=f32)
        for g in range(7)
    ]
    a1 = jnp.concatenate(parts, axis=1)
    a1_s[...] = jnp.maximum(a1 + b1_ref[...], 0.0).astype(_DT)
    a3 = jnp.dot(a1_s[...], w3_ref[...], preferred_element_type=f32)
    a3_s[...] = jnp.maximum(a3 + b3_ref[...], 0.0).astype(_DT)
    y = jnp.dot(a3_s[...], w5_ref[...], preferred_element_type=f32)
    out_ref[...] = y + bo_ref[...]


def _prep_weights(B1, bb1, S2L, B3f, bb3, S4L, W5o):
    kk = np.arange(8)
    q = np.arange(4)
    diff = kk[:, None] - q[None, :]
    band = (diff >= 0) & (diff < 5)
    B1p = jnp.pad(B1, ((0, 0), (0, 4), (0, 0)))
    g = B1p[np.clip(diff, 0, 4)]
    Wg = jnp.where(band[:, :, None, None], g, 0.0)
    Wg = Wg.transpose(0, 2, 1, 3).reshape(256, 448)
    Wg = jnp.pad(Wg, ((0, 0), (0, 64)))
    b1x = jnp.tile(jnp.pad(jnp.tile(bb1, (1, 4)), ((0, 0), (0, 64))),
                   (1, 7))

    taps = np.arange(10)[None, :] + np.arange(5)[:, None]
    S2g = S2L[taps]
    W3 = jnp.einsum("dqr,dcm->rcqm", S2g, B3f)
    W3 = W3.reshape(7, 448, 1600)
    W3 = jnp.pad(W3, ((0, 0), (0, 64), (0, 0))).reshape(3584, 1600)
    b3x = jnp.tile(bb3, (1, 10))

    W5 = jnp.einsum("iq,icm->qcm", S4L, W5o).reshape(1600, 128)
    return (Wg.astype(_DT), b1x, W3.astype(_DT), b3x, W5.astype(_DT))


def kernel(x, B1, bb1, S2L, B3f, bb3, S4L, W5o, bo):
    N = x.shape[0]
    Wg, b1x, W3, b3x, W5 = _prep_weights(B1, bb1, S2L, B3f, bb3, S4L, W5o)

    TB = int(min(1024, _round_up(max(N, 1), 8)))
    Npad = _round_up(N, TB)
    x4 = jnp.pad(x.reshape(N, 28, 28).astype(_DT),
                 ((0, Npad - N), (2, 2), (0, 4))).reshape(Npad, 1024)

    out = pl.pallas_call(
        _lenet_body,
        out_shape=jax.ShapeDtypeStruct((Npad, 128), jnp.float32),
        grid=(Npad // TB,),
        in_specs=[
            pl.BlockSpec((TB, 1024), lambda n: (n, 0)),
            pl.BlockSpec((256, 512), lambda n: (0, 0)),
            pl.BlockSpec((1, 3584), lambda n: (0, 0)),
            pl.BlockSpec((3584, 1600), lambda n: (0, 0)),
            pl.BlockSpec((1, 1600), lambda n: (0, 0)),
            pl.BlockSpec((1600, 128), lambda n: (0, 0)),
            pl.BlockSpec((1, 128), lambda n: (0, 0)),
        ],
        out_specs=pl.BlockSpec((TB, 128), lambda n: (n, 0)),
        scratch_shapes=[
            pltpu.VMEM((TB, 3584), _DT),
            pltpu.VMEM((TB, 1600), _DT),
        ],
        compiler_params=pltpu.CompilerParams(
            dimension_semantics=("parallel",)),
    )(x4, Wg, b1x, W3, b3x, W5, bo)

    return out[:N, :10]


# speedup vs baseline: 1.1320x; 1.1320x over previous
import numpy as np

import jax
import jax.numpy as jnp
from jax.experimental import pallas as pl
from jax.experimental.pallas import tpu as pltpu


_DT = jnp.bfloat16


def _round_up(v, m):
    return (v + m - 1) // m * m


def _lenet_body(x_ref, w1_ref, b1_ref, w3_ref, b3_ref, w5_ref, bo_ref,
                out_ref, a1_s, a3_s):
    f32 = jnp.float32
    xb = x_ref[...].astype(_DT)
    a1 = jnp.dot(xb, w1_ref[...], preferred_element_type=f32)
    a1_s[...] = jnp.maximum(a1 + b1_ref[...], 0.0).astype(_DT)
    a3 = jnp.dot(a1_s[...], w3_ref[...], preferred_element_type=f32)
    a3_s[...] = jnp.maximum(a3 + b3_ref[...], 0.0).astype(_DT)
    y = jnp.dot(a3_s[...], w5_ref[...], preferred_element_type=f32)
    out_ref[...] = y + bo_ref[...]


def _prep_weights(B1, bb1, S2L, B3f, bb3, S4L, W5o):
    k2 = np.arange(28)
    r = np.arange(28)
    diff = k2[:, None] - r[None, :] + 2
    band = (diff >= 0) & (diff < 5)
    g = B1[np.clip(diff, 0, 4)]
    W1 = jnp.where(band[:, :, None, None], g, 0.0)
    W1 = W1.transpose(0, 2, 1, 3).reshape(784, 3136)
    b1x = jnp.tile(bb1, (1, 28))

    taps = np.arange(10)[None, :] + np.arange(5)[:, None]
    S2g = S2L[taps]
    W3 = jnp.einsum("dqr,dcm->rcqm", S2g, B3f).reshape(3136, 1600)
    b3x = jnp.tile(bb3, (1, 10))

    W5 = jnp.einsum("iq,icm->qcm", S4L, W5o).reshape(1600, 128)
    return (W1.astype(_DT), b1x, W3.astype(_DT), b3x, W5.astype(_DT))


def kernel(x, B1, bb1, S2L, B3f, bb3, S4L, W5o, bo):
    N = x.shape[0]
    W1, b1x, W3, b3x, W5 = _prep_weights(B1, bb1, S2L, B3f, bb3, S4L, W5o)

    TB = int(min(1024, _round_up(max(N, 1), 8)))
    Npad = _round_up(N, TB)
    xf = x.reshape(N, 784)
    if Npad != N:
        xf = jnp.pad(xf, ((0, Npad - N), (0, 0)))

    out = pl.pallas_call(
        _lenet_body,
        out_shape=jax.ShapeDtypeStruct((Npad, 128), jnp.float32),
        grid=(Npad // TB,),
        in_specs=[
            pl.BlockSpec((TB, 784), lambda n: (n, 0)),
            pl.BlockSpec((784, 3136), lambda n: (0, 0)),
            pl.BlockSpec((1, 3136), lambda n: (0, 0)),
            pl.BlockSpec((3136, 1600), lambda n: (0, 0)),
            pl.BlockSpec((1, 1600), lambda n: (0, 0)),
            pl.BlockSpec((1600, 128), lambda n: (0, 0)),
            pl.BlockSpec((1, 128), lambda n: (0, 0)),
        ],
        out_specs=pl.BlockSpec((TB, 128), lambda n: (n, 0)),
        scratch_shapes=[
            pltpu.VMEM((TB, 3136), _DT),
            pltpu.VMEM((TB, 1600), _DT),
        ],
        compiler_params=pltpu.CompilerParams(
            dimension_semantics=("parallel",)),
    )(xf, W1, b1x, W3, b3x, W5, bo)

    return out[:N, :10]
